# baseline (speedup 1.0000x reference)
import jax
import jax.numpy as jnp
import numpy as np
from jax.experimental import pallas as pl
from jax.experimental.pallas import tpu as pltpu

B, C, GD, GH, GW = 8, 12, 8, 16, 16
H, W = 512, 512
HB = 256
SB = 16
NS = HB // SB
NJ = H // HB


def _interp_matrix(npix, ncell):
    w = np.arange(npix)
    g = (w + 0.5) * ncell / npix - 0.5
    f = np.floor(g).astype(np.int64)
    w1 = (g - f).astype(np.float32)
    w0 = 1.0 - w1
    m = np.zeros((ncell, npix), np.float32)
    np.add.at(m, (np.clip(f, 0, ncell - 1), w), w0)
    np.add.at(m, (np.clip(f + 1, 0, ncell - 1), w), w1)
    return m


def _body(g5_ref, rxt_ref, guide_ref, out_ref, u_ref):
    j = pl.program_id(1)

    @pl.when(j == 0)
    def _():
        u = jnp.dot(g5_ref[0], rxt_ref[...],
                    preferred_element_type=jnp.float32)
        ub = u.astype(jnp.bfloat16).reshape(GH, C, GD, 1, W)
        u_ref[...] = jnp.broadcast_to(ub, (GH, C, GD, SB, W))

    for s in range(NS):
        band = j * NS + s
        fy = (band - 1) // 2
        yi0 = jnp.clip(fy, 0, GH - 1)
        yi1 = jnp.clip(fy + 1, 0, GH - 1)
        hrow = (jax.lax.broadcasted_iota(jnp.int32, (SB, 1), 0)
                + band * SB).astype(jnp.float32)
        gy = (hrow + 0.5) * (GH / H) - 0.5
        wy1 = (gy - fy.astype(jnp.float32)).astype(jnp.bfloat16)
        wy0 = (1.0 - wy1.astype(jnp.float32)).astype(jnp.bfloat16)

        u0 = u_ref[yi0]
        u1 = u_ref[yi1]

        g = guide_ref[0, s * SB:(s + 1) * SB]
        t = jnp.clip(g * GD - 0.5, 0.0, GD - 1.0)
        wz = [jnp.maximum(1.0 - jnp.abs(t - k), 0.0).astype(jnp.bfloat16)
              for k in range(GD)]

        for c in range(C):
            a0 = wz[0] * u0[c, 0]
            a1 = wz[0] * u1[c, 0]
            for k in range(1, GD):
                a0 = a0 + wz[k] * u0[c, k]
                a1 = a1 + wz[k] * u1[c, k]
            out_ref[0, c, s * SB:(s + 1) * SB] = (
                wy0 * a0 + wy1 * a1).astype(jnp.float32)


@jax.jit
def kernel(bilateral_grid, guidemap):
    g5 = jnp.transpose(bilateral_grid, (0, 3, 1, 2, 4)).reshape(B, GH * C * GD, GW)
    rxt = jnp.asarray(_interp_matrix(W, GW))
    guide = guidemap.reshape(B, H, W)

    return pl.pallas_call(
        _body,
        grid=(B, NJ),
        in_specs=[
            pl.BlockSpec((1, GH * C * GD, GW), lambda b, j: (b, 0, 0)),
            pl.BlockSpec((GW, W), lambda b, j: (0, 0)),
            pl.BlockSpec((1, HB, W), lambda b, j: (b, j, 0)),
        ],
        out_specs=pl.BlockSpec((1, C, HB, W), lambda b, j: (b, 0, j, 0)),
        out_shape=jax.ShapeDtypeStruct((B, C, H, W), jnp.float32),
        scratch_shapes=[pltpu.VMEM((GH, C, GD, SB, W), jnp.bfloat16)],
    )(g5, rxt, guide)


# speedup vs baseline: 1.0011x; 1.0011x over previous
import jax
import jax.numpy as jnp
import numpy as np
from jax.experimental import pallas as pl
from jax.experimental.pallas import tpu as pltpu

B, C, GD, GH, GW = 8, 12, 8, 16, 16
H, W = 512, 512
HB = 128
SB = 16
NS = HB // SB
NJ = H // HB


def _interp_matrix(npix, ncell):
    w = np.arange(npix)
    g = (w + 0.5) * ncell / npix - 0.5
    f = np.floor(g).astype(np.int64)
    w1 = (g - f).astype(np.float32)
    w0 = 1.0 - w1
    m = np.zeros((ncell, npix), np.float32)
    np.add.at(m, (np.clip(f, 0, ncell - 1), w), w0)
    np.add.at(m, (np.clip(f + 1, 0, ncell - 1), w), w1)
    return m


def _body(g5_ref, rxt_ref, guide_ref, out_ref, u_ref):
    j = pl.program_id(1)

    @pl.when(j == 0)
    def _():
        u = jnp.dot(g5_ref[0], rxt_ref[...],
                    preferred_element_type=jnp.float32)
        ub = u.astype(jnp.bfloat16).reshape(GH, C, GD, 1, W)
        u_ref[...] = jnp.broadcast_to(ub, (GH, C, GD, SB, W))

    for s in range(NS):
        band = j * NS + s
        fy = (band - 1) // 2
        yi0 = jnp.clip(fy, 0, GH - 1)
        yi1 = jnp.clip(fy + 1, 0, GH - 1)
        hrow = (jax.lax.broadcasted_iota(jnp.int32, (SB, 1), 0)
                + band * SB).astype(jnp.float32)
        gy = (hrow + 0.5) * (GH / H) - 0.5
        wy1 = (gy - fy.astype(jnp.float32)).astype(jnp.bfloat16)
        wy0 = (1.0 - wy1.astype(jnp.float32)).astype(jnp.bfloat16)

        u0 = u_ref[yi0]
        u1 = u_ref[yi1]

        g = guide_ref[0, s * SB:(s + 1) * SB]
        t = jnp.clip(g * GD - 0.5, 0.0, GD - 1.0)
        wz = [jnp.maximum(1.0 - jnp.abs(t - k), 0.0).astype(jnp.bfloat16)
              for k in range(GD)]

        for c in range(C):
            a0 = wz[0] * u0[c, 0]
            a1 = wz[0] * u1[c, 0]
            for k in range(1, GD):
                a0 = a0 + wz[k] * u0[c, k]
                a1 = a1 + wz[k] * u1[c, k]
            out_ref[0, c, s * SB:(s + 1) * SB] = (
                wy0 * a0 + wy1 * a1).astype(jnp.float32)


@jax.jit
def kernel(bilateral_grid, guidemap):
    g5 = jnp.transpose(bilateral_grid, (0, 3, 1, 2, 4)).reshape(B, GH * C * GD, GW)
    rxt = jnp.asarray(_interp_matrix(W, GW))
    guide = guidemap.reshape(B, H, W)

    return pl.pallas_call(
        _body,
        grid=(B, NJ),
        in_specs=[
            pl.BlockSpec((1, GH * C * GD, GW), lambda b, j: (b, 0, 0)),
            pl.BlockSpec((GW, W), lambda b, j: (0, 0)),
            pl.BlockSpec((1, HB, W), lambda b, j: (b, j, 0)),
        ],
        out_specs=pl.BlockSpec((1, C, HB, W), lambda b, j: (b, 0, j, 0)),
        out_shape=jax.ShapeDtypeStruct((B, C, H, W), jnp.float32),
        scratch_shapes=[pltpu.VMEM((GH, C, GD, SB, W), jnp.bfloat16)],
    )(g5, rxt, guide)
